# baseline (speedup 1.0000x reference)
import functools

import jax
import jax.numpy as jnp
from jax import lax
from jax.experimental import pallas as pl
from jax.experimental.pallas import tpu as pltpu
from jax.experimental.pallas import tpu_sc as plsc

B = 16384
D = 64
H = 32

NC = 2
NS = 16
NW = NC * NS
BPW = B // NW
CHUNK = 128
NCH = BPW // CHUNK

TILE = 2048


def _sc_gather_body(uidx_h, iidx_h, lidx_h, cidx_h,
                    uemb, iemb, lemb, cemb,
                    u_out, i_out, l_out, c_out,
                    uidx_v, iidx_v, lidx_v, cidx_v,
                    urows, irows, lrows, crows, sem):
  wid = lax.axis_index("s") * NC + lax.axis_index("c")
  base = wid * BPW
  row = wid * NCH
  pltpu.sync_copy(uidx_h.at[pl.ds(row, NCH)], uidx_v)
  pltpu.sync_copy(iidx_h.at[pl.ds(row, NCH)], iidx_v)
  pltpu.sync_copy(lidx_h.at[pl.ds(row, NCH)], lidx_v)
  pltpu.sync_copy(cidx_h.at[pl.ds(row, NCH)], cidx_v)
  copies = []
  for j in range(NCH):
    sl = pl.ds(j * CHUNK, CHUNK)
    copies.append(pltpu.async_copy(uemb.at[uidx_v.at[j]], urows.at[sl], sem))
    copies.append(pltpu.async_copy(iemb.at[iidx_v.at[j]], irows.at[sl], sem))
    copies.append(pltpu.async_copy(lemb.at[lidx_v.at[j]], lrows.at[sl], sem))
    copies.append(pltpu.async_copy(cemb.at[cidx_v.at[j]], crows.at[sl], sem))
  for cp in copies:
    cp.wait()
  pltpu.sync_copy(urows, u_out.at[pl.ds(base, BPW)])
  pltpu.sync_copy(irows, i_out.at[pl.ds(base, BPW)])
  pltpu.sync_copy(lrows, l_out.at[pl.ds(base, BPW)])
  pltpu.sync_copy(crows, c_out.at[pl.ds(base, BPW)])


_sc_gather = functools.partial(
    pl.kernel,
    out_type=(
        jax.ShapeDtypeStruct((B, D), jnp.float32),
        jax.ShapeDtypeStruct((B, D), jnp.float32),
        jax.ShapeDtypeStruct((B, H), jnp.float32),
        jax.ShapeDtypeStruct((B, H), jnp.float32),
    ),
    mesh=plsc.VectorSubcoreMesh(core_axis_name="c", subcore_axis_name="s"),
    scratch_types=[
        pltpu.VMEM((NCH, CHUNK), jnp.int32),
        pltpu.VMEM((NCH, CHUNK), jnp.int32),
        pltpu.VMEM((NCH, CHUNK), jnp.int32),
        pltpu.VMEM((NCH, CHUNK), jnp.int32),
        pltpu.VMEM((BPW, D), jnp.float32),
        pltpu.VMEM((BPW, D), jnp.float32),
        pltpu.VMEM((BPW, H), jnp.float32),
        pltpu.VMEM((BPW, H), jnp.float32),
        pltpu.SemaphoreType.DMA,
    ],
    compiler_params=pltpu.CompilerParams(use_tc_tiling_on_sc=False),
)(_sc_gather_body)


def _mlp_body(u_ref, i_ref, l_ref, c_ref, cwi_ref, cwl_ref, cwc_ref, cb_ref,
              w1u_ref, w1c_ref, b1_ref, w2t_ref, b2_ref, w3t_ref, b3_ref,
              out_ref):
  ic = i_ref[...] @ cwi_ref[...]
  ic += l_ref[...] @ cwl_ref[...]
  ic += c_ref[...] @ cwc_ref[...]
  ic = jnp.maximum(ic + cb_ref[...], 0.0)
  h1 = u_ref[...] @ w1u_ref[...]
  h1 += ic @ w1c_ref[...]
  h1 = jnp.maximum(h1 + b1_ref[...], 0.0)
  h2 = jnp.maximum(h1 @ w2t_ref[...] + b2_ref[...], 0.0)
  out_ref[...] = h2 @ w3t_ref[...] + b3_ref[...]


def _full(shape):
  return pl.BlockSpec(shape, lambda i: (0, 0))


_mlp = pl.pallas_call(
    _mlp_body,
    grid=(B // TILE,),
    in_specs=[
        pl.BlockSpec((TILE, D), lambda i: (i, 0)),
        pl.BlockSpec((TILE, D), lambda i: (i, 0)),
        pl.BlockSpec((TILE, H), lambda i: (i, 0)),
        pl.BlockSpec((TILE, H), lambda i: (i, 0)),
        _full((D, D)),
        _full((H, D)),
        _full((H, D)),
        _full((1, D)),
        _full((D, 2 * D)),
        _full((D, 2 * D)),
        _full((1, 2 * D)),
        _full((2 * D, D)),
        _full((1, D)),
        _full((D, 1)),
        _full((1, 1)),
    ],
    out_specs=pl.BlockSpec((TILE, 1), lambda i: (i, 0)),
    out_shape=jax.ShapeDtypeStruct((B, 1), jnp.float32),
    compiler_params=pltpu.CompilerParams(
        dimension_semantics=("arbitrary",)),
)


def kernel(user, item, language, category, user_emb, item_emb, language_emb,
           category_emb, cw, cb, w1, b1, w2, b2, w3, b3):
  u_rows, i_rows, l_rows, c_rows = _sc_gather(
      user.reshape(B // CHUNK, CHUNK),
      item.reshape(B // CHUNK, CHUNK),
      language.reshape(B // CHUNK, CHUNK),
      category.reshape(B // CHUNK, CHUNK),
      user_emb, item_emb, language_emb, category_emb)
  cwi = cw[:, :D].T
  cwl = cw[:, D:D + H].T
  cwc = cw[:, D + H:].T
  w1u = w1[:, :D].T
  w1c = w1[:, D:].T
  out = _mlp(u_rows, i_rows, l_rows, c_rows,
             cwi, cwl, cwc, cb.reshape(1, D),
             w1u, w1c, b1.reshape(1, 2 * D),
             w2.T, b2.reshape(1, D),
             w3.T, b3.reshape(1, 1))
  return out[:, 0]


# speedup vs baseline: 1.5587x; 1.5587x over previous
import functools

import jax
import jax.numpy as jnp
from jax import lax
from jax.experimental import pallas as pl
from jax.experimental.pallas import tpu as pltpu
from jax.experimental.pallas import tpu_sc as plsc

B = 16384
NU = 1000000
NI = 100000
NL = 100
NCAT = 1000
D = 64
H = 32

NC = 2
NS = 16
NW = NC * NS
BPW = B // NW

TILE = 512


def _sc_gather_body(uidx_h, iidx_h, uemb, iemb, u_out, i_out,
                    idx_v, rows_v, sem):
  wid = lax.axis_index("s") * NC + lax.axis_index("c")
  base = wid * BPW
  lanes = lax.iota(jnp.int32, 16)
  for idx_h, tab, out in ((uidx_h, uemb, u_out), (iidx_h, iemb, i_out)):
    pltpu.sync_copy(idx_h.at[wid], idx_v)

    def fire(g, carry):
      vec = idx_v[pl.ds(g * 16, 16)]
      for j in range(16):
        r = jnp.sum(jnp.where(lanes == j, vec, 0))
        pltpu.async_copy(
            tab.at[pl.ds(r, 1)], rows_v.at[pl.ds(g * 16 + j, 1)], sem)
      return carry

    lax.fori_loop(0, BPW // 16, fire, 0)
    pltpu.make_async_copy(tab.at[pl.ds(0, BPW)], rows_v, sem).wait()
    pltpu.sync_copy(rows_v, out.at[pl.ds(base, BPW)])


_sc_gather = functools.partial(
    pl.kernel,
    out_type=(
        jax.ShapeDtypeStruct((B, D), jnp.float32),
        jax.ShapeDtypeStruct((B, D), jnp.float32),
    ),
    mesh=plsc.VectorSubcoreMesh(core_axis_name="c", subcore_axis_name="s"),
    scratch_types=[
        pltpu.VMEM((BPW,), jnp.int32),
        pltpu.VMEM((BPW, D), jnp.float32),
        pltpu.SemaphoreType.DMA,
    ],
    compiler_params=pltpu.CompilerParams(needs_layout_passes=False),
)(_sc_gather_body)


def _mlp_body(u_ref, i_ref, lg_ref, ct_ref,
              lemb_ref, cemb_ref, cwi_ref, cwl_ref, cwc_ref, cb_ref,
              w1u_ref, w1c_ref, b1_ref, w2t_ref, b2_ref, w3t_ref, b3_ref,
              out_ref):
  lw = lemb_ref[...] @ cwl_ref[...]
  cw2 = cemb_ref[...] @ cwc_ref[...]
  ohl = (lg_ref[...] == lax.broadcasted_iota(jnp.int32, (1, NL), 1)
         ).astype(jnp.float32)
  ohc = (ct_ref[...] == lax.broadcasted_iota(jnp.int32, (1, NCAT), 1)
         ).astype(jnp.float32)
  ic = i_ref[...] @ cwi_ref[...]
  ic += ohl @ lw
  ic += ohc @ cw2
  ic = jnp.maximum(ic + cb_ref[...], 0.0)
  h1 = u_ref[...] @ w1u_ref[...]
  h1 += ic @ w1c_ref[...]
  h1 = jnp.maximum(h1 + b1_ref[...], 0.0)
  h2 = jnp.maximum(h1 @ w2t_ref[...] + b2_ref[...], 0.0)
  out_ref[...] = h2 @ w3t_ref[...] + b3_ref[...]


def _full(shape):
  return pl.BlockSpec(shape, lambda i: tuple(0 for _ in shape))


_mlp = pl.pallas_call(
    _mlp_body,
    grid=(B // TILE,),
    in_specs=[
        pl.BlockSpec((TILE, D), lambda i: (i, 0)),
        pl.BlockSpec((TILE, D), lambda i: (i, 0)),
        pl.BlockSpec((TILE, 1), lambda i: (i, 0)),
        pl.BlockSpec((TILE, 1), lambda i: (i, 0)),
        _full((NL, H)),
        _full((NCAT, H)),
        _full((D, D)),
        _full((H, D)),
        _full((H, D)),
        _full((1, D)),
        _full((D, 2 * D)),
        _full((D, 2 * D)),
        _full((1, 2 * D)),
        _full((2 * D, D)),
        _full((1, D)),
        _full((D, 1)),
        _full((1, 1)),
    ],
    out_specs=pl.BlockSpec((TILE, 1), lambda i: (i, 0)),
    out_shape=jax.ShapeDtypeStruct((B, 1), jnp.float32),
    compiler_params=pltpu.CompilerParams(
        dimension_semantics=("arbitrary",)),
)


def kernel(user, item, language, category, user_emb, item_emb, language_emb,
           category_emb, cw, cb, w1, b1, w2, b2, w3, b3):
  u_rows, i_rows = _sc_gather(
      user.reshape(NW, BPW), item.reshape(NW, BPW), user_emb, item_emb)
  cwi = cw[:, :D].T
  cwl = cw[:, D:D + H].T
  cwc = cw[:, D + H:].T
  w1u = w1[:, :D].T
  w1c = w1[:, D:].T
  out = _mlp(u_rows, i_rows,
             language.reshape(B, 1), category.reshape(B, 1),
             language_emb, category_emb,
             cwi, cwl, cwc, cb.reshape(1, D),
             w1u, w1c, b1.reshape(1, 2 * D),
             w2.T, b2.reshape(1, D),
             w3.T, b3.reshape(1, 1))
  return out[:, 0]
